# baseline (speedup 1.0000x reference)
import functools

import jax
import jax.numpy as jnp
from jax import lax
from jax.experimental import pallas as pl
from jax.experimental.pallas import tpu as pltpu
from jax.experimental.pallas import tpu_sc as plsc

N_EDGES = 320000
D_FEAT = 128
N_SEGMENTS = 10000

NUM_CORES = 2
NUM_SUBCORES = 16
HALF = D_FEAT // NUM_CORES

ROWS_PER_SCATTER = 100
IDROWS = N_EDGES // ROWS_PER_SCATTER
ROWS_PER_TILE = IDROWS // NUM_SUBCORES
CHUNK_IDROWS = 5
CHUNK_EDGES = CHUNK_IDROWS * ROWS_PER_SCATTER
N_CHUNKS = ROWS_PER_TILE // CHUNK_IDROWS
N_PAIRS = N_CHUNKS // 2
SEG_PER_TILE = N_SEGMENTS // NUM_SUBCORES


def _sc_body(data_hbm, seg_hbm, zero_hbm, out_hbm, acc,
             idx0, idx1, d0, d1, sem0, sem1, ssem0, ssem1):
    c = lax.axis_index("c")
    s = lax.axis_index("s")
    col0 = c * HALF
    seg0 = s * SEG_PER_TILE

    row_base = s * ROWS_PER_TILE

    def src_slices(chunk):
        r = row_base + chunk * CHUNK_IDROWS
        return (seg_hbm.at[pl.ds(r, CHUNK_IDROWS)],
                data_hbm.at[pl.ds(r * ROWS_PER_SCATTER, CHUNK_EDGES),
                            pl.ds(col0, HALF)])

    def start_load(chunk, idx_v, data_v, sem):
        seg_src, data_src = src_slices(chunk)
        pltpu.async_copy(seg_src, idx_v, sem)
        pltpu.async_copy(data_src, data_v, sem)

    def wait_load(chunk, idx_v, data_v, sem):
        seg_src, data_src = src_slices(chunk)
        pltpu.make_async_copy(seg_src, idx_v, sem).wait()
        pltpu.make_async_copy(data_src, data_v, sem).wait()

    def fire_scatter(idx_v, data_v, sem):
        for j in range(CHUNK_IDROWS):
            pltpu.async_copy(
                data_v.at[pl.ds(j * ROWS_PER_SCATTER, ROWS_PER_SCATTER)],
                acc.at[idx_v.at[j]], sem, add=True)

    def drain_scatter(idx_v, data_v, sem):
        for j in range(CHUNK_IDROWS):
            pltpu.make_async_copy(
                data_v.at[pl.ds(j * ROWS_PER_SCATTER, ROWS_PER_SCATTER)],
                acc.at[idx_v.at[j]], sem).wait()

    start_load(0, idx0, d0, sem0)
    start_load(1, idx1, d1, sem1)
    pltpu.sync_copy(zero_hbm.at[pl.ds(seg0, SEG_PER_TILE)],
                    acc.at[pl.ds(seg0, SEG_PER_TILE)])
    plsc.subcore_barrier()

    def pair_body(k, carry):
        c0 = 2 * k
        wait_load(c0, idx0, d0, sem0)
        fire_scatter(idx0, d0, ssem0)
        wait_load(c0 + 1, idx1, d1, sem1)
        fire_scatter(idx1, d1, ssem1)

        @pl.when(k < N_PAIRS - 1)
        def _():
            drain_scatter(idx0, d0, ssem0)
            start_load(c0 + 2, idx0, d0, sem0)
            drain_scatter(idx1, d1, ssem1)
            start_load(c0 + 3, idx1, d1, sem1)

        @pl.when(k == N_PAIRS - 1)
        def _():
            drain_scatter(idx0, d0, ssem0)
            drain_scatter(idx1, d1, ssem1)

        return carry

    lax.fori_loop(0, N_PAIRS, pair_body, 0)
    plsc.subcore_barrier()

    pltpu.sync_copy(acc.at[pl.ds(seg0, SEG_PER_TILE)],
                    out_hbm.at[pl.ds(seg0, SEG_PER_TILE), pl.ds(col0, HALF)])


@functools.partial(jax.jit, static_argnames=())
def _segment_sum_sc(data, seg2d, zeros):
    mesh = plsc.VectorSubcoreMesh(core_axis_name="c", subcore_axis_name="s")
    return pl.kernel(
        _sc_body,
        out_type=jax.ShapeDtypeStruct((N_SEGMENTS, D_FEAT), jnp.float32),
        mesh=mesh,
        scratch_types=[
            pltpu.MemorySpace.VMEM_SHARED((N_SEGMENTS, HALF), jnp.float32),
            pltpu.VMEM((CHUNK_IDROWS, ROWS_PER_SCATTER), jnp.int32),
            pltpu.VMEM((CHUNK_IDROWS, ROWS_PER_SCATTER), jnp.int32),
            pltpu.VMEM((CHUNK_EDGES, HALF), jnp.float32),
            pltpu.VMEM((CHUNK_EDGES, HALF), jnp.float32),
            pltpu.SemaphoreType.DMA,
            pltpu.SemaphoreType.DMA,
            pltpu.SemaphoreType.DMA,
            pltpu.SemaphoreType.DMA,
        ],
        compiler_params=pltpu.CompilerParams(use_tc_tiling_on_sc=False),
    )(data, seg2d, zeros)


def kernel(data, segment_ids):
    seg2d = segment_ids.astype(jnp.int32).reshape(IDROWS, ROWS_PER_SCATTER)
    zeros = jnp.zeros((N_SEGMENTS, HALF), jnp.float32)
    return _segment_sum_sc(data, seg2d, zeros)


# speedup vs baseline: 1.0121x; 1.0121x over previous
import functools

import jax
import jax.numpy as jnp
from jax import lax
from jax.experimental import pallas as pl
from jax.experimental.pallas import tpu as pltpu
from jax.experimental.pallas import tpu_sc as plsc

N_EDGES = 320000
D_FEAT = 128
N_SEGMENTS = 10000

NUM_CORES = 2
NUM_SUBCORES = 16
HALF = D_FEAT // NUM_CORES

ROWS_PER_SCATTER = 100
IDROWS = N_EDGES // ROWS_PER_SCATTER
ROWS_PER_TILE = IDROWS // NUM_SUBCORES
CHUNK_IDROWS = 4
CHUNK_EDGES = CHUNK_IDROWS * ROWS_PER_SCATTER
N_CHUNKS = ROWS_PER_TILE // CHUNK_IDROWS
N_PAIRS = N_CHUNKS // 2
SEG_PER_TILE = N_SEGMENTS // NUM_SUBCORES


def _sc_body(data_hbm, seg_hbm, zero_hbm, out_hbm, acc,
             idx0, idx1, d0, d1, sem0, sem1, ssem0, ssem1):
    c = lax.axis_index("c")
    s = lax.axis_index("s")
    col0 = c * HALF
    seg0 = s * SEG_PER_TILE

    row_base = s * ROWS_PER_TILE

    def src_slices(chunk):
        r = row_base + chunk * CHUNK_IDROWS
        return (seg_hbm.at[pl.ds(r, CHUNK_IDROWS)],
                data_hbm.at[pl.ds(r * ROWS_PER_SCATTER, CHUNK_EDGES),
                            pl.ds(col0, HALF)])

    def start_load(chunk, idx_v, data_v, sem):
        seg_src, data_src = src_slices(chunk)
        pltpu.async_copy(seg_src, idx_v, sem)
        pltpu.async_copy(data_src, data_v, sem)

    def wait_load(chunk, idx_v, data_v, sem):
        seg_src, data_src = src_slices(chunk)
        pltpu.make_async_copy(seg_src, idx_v, sem).wait()
        pltpu.make_async_copy(data_src, data_v, sem).wait()

    def fire_scatter(idx_v, data_v, sem):
        for j in range(CHUNK_IDROWS):
            pltpu.async_copy(
                data_v.at[pl.ds(j * ROWS_PER_SCATTER, ROWS_PER_SCATTER)],
                acc.at[idx_v.at[j]], sem, add=True)

    def drain_scatter(idx_v, data_v, sem):
        for j in range(CHUNK_IDROWS):
            pltpu.make_async_copy(
                data_v.at[pl.ds(j * ROWS_PER_SCATTER, ROWS_PER_SCATTER)],
                acc.at[idx_v.at[j]], sem).wait()

    start_load(0, idx0, d0, sem0)
    start_load(1, idx1, d1, sem1)
    pltpu.sync_copy(zero_hbm.at[pl.ds(seg0, SEG_PER_TILE)],
                    acc.at[pl.ds(seg0, SEG_PER_TILE)])
    plsc.subcore_barrier()

    def pair_body(k, carry):
        c0 = 2 * k
        wait_load(c0, idx0, d0, sem0)
        fire_scatter(idx0, d0, ssem0)
        wait_load(c0 + 1, idx1, d1, sem1)
        fire_scatter(idx1, d1, ssem1)

        @pl.when(k < N_PAIRS - 1)
        def _():
            drain_scatter(idx0, d0, ssem0)
            start_load(c0 + 2, idx0, d0, sem0)
            drain_scatter(idx1, d1, ssem1)
            start_load(c0 + 3, idx1, d1, sem1)

        @pl.when(k == N_PAIRS - 1)
        def _():
            drain_scatter(idx0, d0, ssem0)
            drain_scatter(idx1, d1, ssem1)

        return carry

    lax.fori_loop(0, N_PAIRS, pair_body, 0)
    plsc.subcore_barrier()

    pltpu.sync_copy(acc.at[pl.ds(seg0, SEG_PER_TILE)],
                    out_hbm.at[pl.ds(seg0, SEG_PER_TILE), pl.ds(col0, HALF)])


@functools.partial(jax.jit, static_argnames=())
def _segment_sum_sc(data, seg2d, zeros):
    mesh = plsc.VectorSubcoreMesh(core_axis_name="c", subcore_axis_name="s")
    return pl.kernel(
        _sc_body,
        out_type=jax.ShapeDtypeStruct((N_SEGMENTS, D_FEAT), jnp.float32),
        mesh=mesh,
        scratch_types=[
            pltpu.MemorySpace.VMEM_SHARED((N_SEGMENTS, HALF), jnp.float32),
            pltpu.VMEM((CHUNK_IDROWS, ROWS_PER_SCATTER), jnp.int32),
            pltpu.VMEM((CHUNK_IDROWS, ROWS_PER_SCATTER), jnp.int32),
            pltpu.VMEM((CHUNK_EDGES, HALF), jnp.float32),
            pltpu.VMEM((CHUNK_EDGES, HALF), jnp.float32),
            pltpu.SemaphoreType.DMA,
            pltpu.SemaphoreType.DMA,
            pltpu.SemaphoreType.DMA,
            pltpu.SemaphoreType.DMA,
        ],
        compiler_params=pltpu.CompilerParams(use_tc_tiling_on_sc=False),
    )(data, seg2d, zeros)


def kernel(data, segment_ids):
    seg2d = segment_ids.astype(jnp.int32).reshape(IDROWS, ROWS_PER_SCATTER)
    zeros = jnp.zeros((N_SEGMENTS, HALF), jnp.float32)
    return _segment_sum_sc(data, seg2d, zeros)


# speedup vs baseline: 1.2033x; 1.1889x over previous
import functools

import jax
import jax.numpy as jnp
from jax import lax
from jax.experimental import pallas as pl
from jax.experimental.pallas import tpu as pltpu
from jax.experimental.pallas import tpu_sc as plsc

N_EDGES = 320000
D_FEAT = 128
N_SEGMENTS = 10000

NUM_CORES = 2
NUM_SUBCORES = 16
HALF = D_FEAT // NUM_CORES

ROWS_PER_SCATTER = 100
IDROWS = N_EDGES // ROWS_PER_SCATTER
ROWS_PER_TILE = IDROWS // NUM_SUBCORES
CHUNK_IDROWS = 4
CHUNK_EDGES = CHUNK_IDROWS * ROWS_PER_SCATTER
N_CHUNKS = ROWS_PER_TILE // CHUNK_IDROWS
SEG_PER_TILE = N_SEGMENTS // NUM_SUBCORES


def _sc_body(data_hbm, seg_hbm, zero_hbm, out_hbm, acc,
             idx0, idx1, d0, d1, sem0, sem1, ssem):
    c = lax.axis_index("c")
    s = lax.axis_index("s")
    col0 = c * HALF
    seg0 = s * SEG_PER_TILE

    row_base = s * ROWS_PER_TILE

    def src_slices(chunk):
        r = row_base + chunk * CHUNK_IDROWS
        return (seg_hbm.at[pl.ds(r, CHUNK_IDROWS)],
                data_hbm.at[pl.ds(r * ROWS_PER_SCATTER, CHUNK_EDGES),
                            pl.ds(col0, HALF)])

    def start_load(chunk, idx_v, data_v, sem):
        seg_src, data_src = src_slices(chunk)
        pltpu.async_copy(seg_src, idx_v, sem)
        pltpu.async_copy(data_src, data_v, sem)

    def wait_load(chunk, idx_v, data_v, sem):
        seg_src, data_src = src_slices(chunk)
        pltpu.make_async_copy(seg_src, idx_v, sem).wait()
        pltpu.make_async_copy(data_src, data_v, sem).wait()

    def scatter(idx_v, data_v):
        handles = [
            pltpu.async_copy(
                data_v.at[pl.ds(j * ROWS_PER_SCATTER, ROWS_PER_SCATTER)],
                acc.at[idx_v.at[j]], ssem, add=True)
            for j in range(CHUNK_IDROWS)
        ]
        for h in handles:
            h.wait()

    start_load(0, idx0, d0, sem0)
    start_load(1, idx1, d1, sem1)
    pltpu.sync_copy(zero_hbm.at[pl.ds(seg0, SEG_PER_TILE)],
                    acc.at[pl.ds(seg0, SEG_PER_TILE)])
    plsc.subcore_barrier()

    def pair_body(k, carry):
        c0 = 2 * k
        wait_load(c0, idx0, d0, sem0)
        scatter(idx0, d0)

        @pl.when(k < N_CHUNKS // 2 - 1)
        def _():
            start_load(c0 + 2, idx0, d0, sem0)

        wait_load(c0 + 1, idx1, d1, sem1)
        scatter(idx1, d1)

        @pl.when(k < N_CHUNKS // 2 - 1)
        def _():
            start_load(c0 + 3, idx1, d1, sem1)

        return carry

    lax.fori_loop(0, N_CHUNKS // 2, pair_body, 0)
    plsc.subcore_barrier()

    pltpu.sync_copy(acc.at[pl.ds(seg0, SEG_PER_TILE)],
                    out_hbm.at[pl.ds(seg0, SEG_PER_TILE), pl.ds(col0, HALF)])


@functools.partial(jax.jit, static_argnames=())
def _segment_sum_sc(data, seg2d, zeros):
    mesh = plsc.VectorSubcoreMesh(core_axis_name="c", subcore_axis_name="s")
    return pl.kernel(
        _sc_body,
        out_type=jax.ShapeDtypeStruct((N_SEGMENTS, D_FEAT), jnp.float32),
        mesh=mesh,
        scratch_types=[
            pltpu.MemorySpace.VMEM_SHARED((N_SEGMENTS, HALF), jnp.float32),
            pltpu.VMEM((CHUNK_IDROWS, ROWS_PER_SCATTER), jnp.int32),
            pltpu.VMEM((CHUNK_IDROWS, ROWS_PER_SCATTER), jnp.int32),
            pltpu.VMEM((CHUNK_EDGES, HALF), jnp.float32),
            pltpu.VMEM((CHUNK_EDGES, HALF), jnp.float32),
            pltpu.SemaphoreType.DMA,
            pltpu.SemaphoreType.DMA,
            pltpu.SemaphoreType.DMA,
        ],
        compiler_params=pltpu.CompilerParams(use_tc_tiling_on_sc=False),
    )(data, seg2d, zeros)


def kernel(data, segment_ids):
    seg2d = segment_ids.astype(jnp.int32).reshape(IDROWS, ROWS_PER_SCATTER)
    zeros = jnp.zeros((N_SEGMENTS, HALF), jnp.float32)
    return _segment_sum_sc(data, seg2d, zeros)


# speedup vs baseline: 1.2431x; 1.0330x over previous
import functools

import jax
import jax.numpy as jnp
from jax import lax
from jax.experimental import pallas as pl
from jax.experimental.pallas import tpu as pltpu
from jax.experimental.pallas import tpu_sc as plsc

N_EDGES = 320000
D_FEAT = 128
N_SEGMENTS = 10000

NUM_CORES = 2
NUM_SUBCORES = 16
HALF = D_FEAT // NUM_CORES

EDGES_PER_TILE = N_EDGES // NUM_SUBCORES
ROWS_PER_SCATTER = 128
CHUNK_EDGES = 4 * ROWS_PER_SCATTER
N_FULL = EDGES_PER_TILE // CHUNK_EDGES
TAIL = EDGES_PER_TILE - N_FULL * CHUNK_EDGES
N_PAIRS = N_FULL // 2
SEG_PER_TILE = N_SEGMENTS // NUM_SUBCORES


def _sc_body(data_hbm, seg_hbm, zero_hbm, out_hbm, acc,
             idx0, idx1, idxt, d0, d1, dt, sem0, sem1, ssem):
    c = lax.axis_index("c")
    s = lax.axis_index("s")
    col0 = c * HALF
    seg0 = s * SEG_PER_TILE
    edge_base = s * EDGES_PER_TILE

    def start_load(chunk, idx_v, data_v, sem):
        e = edge_base + chunk * CHUNK_EDGES
        pltpu.async_copy(seg_hbm.at[pl.ds(e, CHUNK_EDGES)], idx_v, sem)
        pltpu.async_copy(
            data_hbm.at[pl.ds(e, CHUNK_EDGES), pl.ds(col0, HALF)],
            data_v, sem)

    def wait_load(chunk, idx_v, data_v, sem):
        e = edge_base + chunk * CHUNK_EDGES
        pltpu.make_async_copy(seg_hbm.at[pl.ds(e, CHUNK_EDGES)], idx_v,
                              sem).wait()
        pltpu.make_async_copy(
            data_hbm.at[pl.ds(e, CHUNK_EDGES), pl.ds(col0, HALF)],
            data_v, sem).wait()

    def scatter(idx_v, data_v):
        handles = [
            pltpu.async_copy(
                data_v.at[pl.ds(j * ROWS_PER_SCATTER, ROWS_PER_SCATTER)],
                acc.at[idx_v.at[pl.ds(j * ROWS_PER_SCATTER,
                                      ROWS_PER_SCATTER)]],
                ssem, add=True)
            for j in range(CHUNK_EDGES // ROWS_PER_SCATTER)
        ]
        for h in handles:
            h.wait()

    start_load(0, idx0, d0, sem0)
    start_load(1, idx1, d1, sem1)
    pltpu.sync_copy(zero_hbm.at[pl.ds(seg0, SEG_PER_TILE)],
                    acc.at[pl.ds(seg0, SEG_PER_TILE)])
    plsc.subcore_barrier()

    def pair_body(k, carry):
        c0 = 2 * k
        wait_load(c0, idx0, d0, sem0)
        scatter(idx0, d0)

        @pl.when(k < N_PAIRS - 1)
        def _():
            start_load(c0 + 2, idx0, d0, sem0)

        wait_load(c0 + 1, idx1, d1, sem1)
        scatter(idx1, d1)

        @pl.when(k < N_PAIRS - 1)
        def _():
            start_load(c0 + 3, idx1, d1, sem1)

        return carry

    lax.fori_loop(0, N_PAIRS, pair_body, 0)

    et = edge_base + N_FULL * CHUNK_EDGES
    pltpu.async_copy(seg_hbm.at[pl.ds(et, TAIL)], idxt, sem1)
    pltpu.async_copy(data_hbm.at[pl.ds(et, TAIL), pl.ds(col0, HALF)],
                     dt, sem1)
    start_load(N_FULL - 1, idx0, d0, sem0)
    wait_load(N_FULL - 1, idx0, d0, sem0)
    scatter(idx0, d0)
    pltpu.make_async_copy(seg_hbm.at[pl.ds(et, TAIL)], idxt, sem1).wait()
    pltpu.make_async_copy(
        data_hbm.at[pl.ds(et, TAIL), pl.ds(col0, HALF)], dt, sem1).wait()
    pltpu.async_copy(dt, acc.at[idxt], ssem, add=True).wait()

    plsc.subcore_barrier()

    pltpu.sync_copy(acc.at[pl.ds(seg0, SEG_PER_TILE)],
                    out_hbm.at[pl.ds(seg0, SEG_PER_TILE), pl.ds(col0, HALF)])


@functools.partial(jax.jit, static_argnames=())
def _segment_sum_sc(data, seg_ids, zeros):
    mesh = plsc.VectorSubcoreMesh(core_axis_name="c", subcore_axis_name="s")
    return pl.kernel(
        _sc_body,
        out_type=jax.ShapeDtypeStruct((N_SEGMENTS, D_FEAT), jnp.float32),
        mesh=mesh,
        scratch_types=[
            pltpu.MemorySpace.VMEM_SHARED((N_SEGMENTS, HALF), jnp.float32),
            pltpu.VMEM((CHUNK_EDGES,), jnp.int32),
            pltpu.VMEM((CHUNK_EDGES,), jnp.int32),
            pltpu.VMEM((TAIL,), jnp.int32),
            pltpu.VMEM((CHUNK_EDGES, HALF), jnp.float32),
            pltpu.VMEM((CHUNK_EDGES, HALF), jnp.float32),
            pltpu.VMEM((TAIL, HALF), jnp.float32),
            pltpu.SemaphoreType.DMA,
            pltpu.SemaphoreType.DMA,
            pltpu.SemaphoreType.DMA,
        ],
        compiler_params=pltpu.CompilerParams(use_tc_tiling_on_sc=False),
    )(data, seg_ids, zeros)


def kernel(data, segment_ids):
    seg_ids = segment_ids.astype(jnp.int32)
    zeros = jnp.zeros((N_SEGMENTS, HALF), jnp.float32)
    return _segment_sum_sc(data, seg_ids, zeros)
